# baseline (speedup 1.0000x reference)
import functools

import jax
import jax.numpy as jnp
from jax.experimental import pallas as pl

_SEQ_BLK = 256
_HALO = 8
_TREE_ITERS = 3
_LN_EPS = 1e-5


def _phase_kernel(xin_ref, halo_ref, wc1_ref, wc2_ref, bias_ref, out_ref,
                  *, reverse, seq):
    j = pl.program_id(1)
    x_blk = xin_ref[0]
    halo = halo_ref[0]
    d = x_blk.shape[1]
    b_edge = bias_ref[0, :]
    b_msg = bias_ref[1, :]
    b_gate = bias_ref[2, :]
    ln_g = bias_ref[3, :]
    ln_b = bias_ref[4, :]

    if reverse:
        hx = jnp.concatenate([x_blk, halo], axis=0)
        row0 = j * _SEQ_BLK
    else:
        hx = jnp.concatenate([halo, x_blk], axis=0)
        row0 = j * _SEQ_BLK - _HALO

    mu = jnp.mean(hx, axis=-1, keepdims=True)
    var = jnp.mean((hx - mu) ** 2, axis=-1, keepdims=True)
    h = (hx - mu) * jax.lax.rsqrt(var + _LN_EPS) * ln_g + ln_b

    gid = row0 + jax.lax.broadcasted_iota(jnp.int32, (h.shape[0], 1), 0)
    bound = (gid == (seq - 1)) if reverse else (gid == 0)
    zrow = jnp.zeros((1, d), dtype=h.dtype)

    wc1 = wc1_ref[...]
    wc2 = wc2_ref[...]
    for _ in range(_TREE_ITERS):
        p = jnp.dot(h.astype(jnp.bfloat16), wc1,
                    preferred_element_type=jnp.float32)
        msg = p[:, :d] + b_edge
        if reverse:
            agg = jnp.concatenate([msg[1:], zrow], axis=0)
        else:
            agg = jnp.concatenate([zrow, msg[:-1]], axis=0)
        agg = jnp.where(bound, 0.0, agg)
        q = jnp.dot(agg.astype(jnp.bfloat16), wc2,
                    preferred_element_type=jnp.float32)
        upd = jnp.tanh(q[:, :d] + b_msg)
        gate = jax.nn.sigmoid(p[:, d:] + q[:, d:] + b_gate)
        h = h + gate * upd

    if reverse:
        out_ref[0] = x_blk + h[:_SEQ_BLK]
    else:
        out_ref[0] = x_blk + h[_HALO:]


def _phase(x_in, p, reverse):
    b, s, d = x_in.shape
    wc1 = jnp.concatenate([p["W_edge"], p["W_gate"][:d]],
                          axis=1).astype(jnp.bfloat16)
    wc2 = jnp.concatenate([p["W_msg"], p["W_gate"][d:]],
                          axis=1).astype(jnp.bfloat16)
    zero = jnp.zeros_like(p["b_edge"])
    bias = jnp.stack([p["b_edge"], p["b_msg"], p["b_gate"],
                      p["ln_g"], p["ln_b"], zero, zero, zero])

    nblk = s // _SEQ_BLK
    hb = _SEQ_BLK // _HALO
    last_halo_blk = s // _HALO - 1

    if reverse:
        def halo_map(bi, ji):
            return (bi, jnp.minimum((ji + 1) * hb, last_halo_blk), 0)
    else:
        def halo_map(bi, ji):
            return (bi, jnp.maximum(ji * hb - 1, 0), 0)

    return pl.pallas_call(
        functools.partial(_phase_kernel, reverse=reverse, seq=s),
        grid=(b, nblk),
        in_specs=[
            pl.BlockSpec((1, _SEQ_BLK, d), lambda bi, ji: (bi, ji, 0)),
            pl.BlockSpec((1, _HALO, d), halo_map),
            pl.BlockSpec((d, 2 * d), lambda bi, ji: (0, 0)),
            pl.BlockSpec((d, 2 * d), lambda bi, ji: (0, 0)),
            pl.BlockSpec((8, d), lambda bi, ji: (0, 0)),
        ],
        out_specs=pl.BlockSpec((1, _SEQ_BLK, d), lambda bi, ji: (bi, ji, 0)),
        out_shape=jax.ShapeDtypeStruct((b, s, d), x_in.dtype),
    )(x_in, x_in, wc1, wc2, bias)


def kernel(x, params):
    h = _phase(x, params["enc"], reverse=False)
    h = _phase(h, params["dec"], reverse=True)
    return h


# speedup vs baseline: 1.0591x; 1.0591x over previous
import functools

import jax
import jax.numpy as jnp
from jax.experimental import pallas as pl

_SEQ_BLK = 512
_HALO = 8
_NCHUNK = 2
_TREE_ITERS = 3
_LN_EPS = 1e-5


def _phase_kernel(xin_ref, halo_ref, wc1_ref, wc2_ref, bias_ref, out_ref,
                  *, reverse, nblk):
    j = pl.program_id(1)
    x_blk = xin_ref[0]
    halo = halo_ref[0]
    d = x_blk.shape[1]
    b_edge = bias_ref[0, :]
    b2_msg = bias_ref[1, :]
    b2_gate = bias_ref[2, :]
    ln_g = bias_ref[3, :]
    ln_b = bias_ref[4, :]

    if reverse:
        hx = jnp.concatenate([x_blk, halo], axis=0)
    else:
        hx = jnp.concatenate([halo, x_blk], axis=0)

    mu = jnp.mean(hx, axis=-1, keepdims=True)
    var = jnp.mean((hx - mu) ** 2, axis=-1, keepdims=True)
    h = (hx - mu) * jax.lax.rsqrt(var + _LN_EPS) * ln_g + ln_b

    iota8 = jax.lax.broadcasted_iota(jnp.int32, (8, 1), 0)
    if reverse:
        fixmask = (iota8 == 7) & (j == nblk - 1)
        fix_at = _SEQ_BLK - 8
    else:
        fixmask = (iota8 == 0) & (j == 0)
        fix_at = _HALO
    zrow = jnp.zeros((1, d), dtype=h.dtype)

    wc1 = wc1_ref[...]
    wc2 = wc2_ref[...]

    m = h.shape[0]
    step = ((m // _NCHUNK) // 8) * 8
    bounds = [0] + [step * i for i in range(1, _NCHUNK)] + [m]
    hs = [h[a:b] for a, b in zip(bounds[:-1], bounds[1:])]
    nc = _NCHUNK

    for _ in range(_TREE_ITERS):
        ps = [jnp.dot(hc.astype(jnp.bfloat16), wc1,
                      preferred_element_type=jnp.float32) for hc in hs]
        pms = [pp[:, :d] for pp in ps]
        aggs = []
        if reverse:
            for c in range(nc):
                nxt = zrow if c == nc - 1 else pms[c + 1][:1]
                if c == nc - 1:
                    off = fix_at - bounds[c]
                    sl = jnp.where(fixmask, -b_edge, pms[c][off + 1:off + 9])
                    aggs.append(jnp.concatenate(
                        [pms[c][1:off + 1], sl, pms[c][off + 9:], nxt],
                        axis=0))
                else:
                    aggs.append(jnp.concatenate([pms[c][1:], nxt], axis=0))
        else:
            for c in range(nc):
                prv = zrow if c == 0 else pms[c - 1][-1:]
                if c == 0:
                    sl = jnp.where(fixmask, -b_edge,
                                   pms[0][fix_at - 1:fix_at + 7])
                    aggs.append(jnp.concatenate(
                        [prv, pms[0][:fix_at - 1], sl, pms[0][fix_at + 7:-1]],
                        axis=0))
                else:
                    aggs.append(jnp.concatenate([prv, pms[c][:-1]], axis=0))
        qs = []
        new_hs = []
        for c in range(nc):
            qs.append(jnp.dot(aggs[c].astype(jnp.bfloat16), wc2,
                              preferred_element_type=jnp.float32))
            if c > 0:
                q, pp = qs[c - 1], ps[c - 1]
                upd = jnp.tanh(q[:, :d] + b2_msg)
                gate = jax.nn.sigmoid(pp[:, d:] + q[:, d:] + b2_gate)
                new_hs.append(hs[c - 1] + gate * upd)
        q, pp = qs[nc - 1], ps[nc - 1]
        upd = jnp.tanh(q[:, :d] + b2_msg)
        gate = jax.nn.sigmoid(pp[:, d:] + q[:, d:] + b2_gate)
        new_hs.append(hs[nc - 1] + gate * upd)
        hs = new_hs

    h = jnp.concatenate(hs, axis=0)
    if reverse:
        out_ref[0] = x_blk + h[:_SEQ_BLK]
    else:
        out_ref[0] = x_blk + h[_HALO:]


def _phase(x_in, p, reverse):
    b, s, d = x_in.shape
    wc1f = jnp.concatenate([p["W_edge"], p["W_gate"][:d]], axis=1)
    wc2f = jnp.concatenate([p["W_msg"], p["W_gate"][d:]], axis=1)
    wc1 = wc1f.astype(jnp.bfloat16)
    wc2 = wc2f.astype(jnp.bfloat16)
    bvec = p["b_edge"] @ wc2f
    zero = jnp.zeros_like(p["b_edge"])
    bias = jnp.stack([p["b_edge"], p["b_msg"] + bvec[:d],
                      p["b_gate"] + bvec[d:],
                      p["ln_g"], p["ln_b"], zero, zero, zero])

    nblk = s // _SEQ_BLK
    hb = _SEQ_BLK // _HALO
    last_halo_blk = s // _HALO - 1

    if reverse:
        def halo_map(bi, ji):
            return (bi, jnp.minimum((ji + 1) * hb, last_halo_blk), 0)
    else:
        def halo_map(bi, ji):
            return (bi, jnp.maximum(ji * hb - 1, 0), 0)

    return pl.pallas_call(
        functools.partial(_phase_kernel, reverse=reverse, nblk=nblk),
        grid=(b, nblk),
        in_specs=[
            pl.BlockSpec((1, _SEQ_BLK, d), lambda bi, ji: (bi, ji, 0)),
            pl.BlockSpec((1, _HALO, d), halo_map),
            pl.BlockSpec((d, 2 * d), lambda bi, ji: (0, 0)),
            pl.BlockSpec((d, 2 * d), lambda bi, ji: (0, 0)),
            pl.BlockSpec((8, d), lambda bi, ji: (0, 0)),
        ],
        out_specs=pl.BlockSpec((1, _SEQ_BLK, d), lambda bi, ji: (bi, ji, 0)),
        out_shape=jax.ShapeDtypeStruct((b, s, d), x_in.dtype),
    )(x_in, x_in, wc1, wc2, bias)


def kernel(x, params):
    h = _phase(x, params["enc"], reverse=False)
    h = _phase(h, params["dec"], reverse=True)
    return h


# speedup vs baseline: 1.0680x; 1.0084x over previous
import functools

import jax
import jax.numpy as jnp
from jax.experimental import pallas as pl

_SEQ_BLK = 512
_HALO = 8
_NCHUNK = 4
_TREE_ITERS = 3
_LN_EPS = 1e-5


def _phase_kernel(xin_ref, halo_ref, wc1_ref, wc2_ref, bias_ref, out_ref,
                  *, reverse, nblk):
    j = pl.program_id(1)
    x_blk = xin_ref[0]
    halo = halo_ref[0]
    d = x_blk.shape[1]
    b_edge = bias_ref[0, :]
    b2_msg = bias_ref[1, :]
    b2_gate = bias_ref[2, :]
    ln_g = bias_ref[3, :]
    ln_b = bias_ref[4, :]

    if reverse:
        hx = jnp.concatenate([x_blk, halo], axis=0)
    else:
        hx = jnp.concatenate([halo, x_blk], axis=0)

    mu = jnp.mean(hx, axis=-1, keepdims=True)
    var = jnp.mean((hx - mu) ** 2, axis=-1, keepdims=True)
    h = (hx - mu) * jax.lax.rsqrt(var + _LN_EPS) * ln_g + ln_b

    iota8 = jax.lax.broadcasted_iota(jnp.int32, (8, 1), 0)
    if reverse:
        fixmask = (iota8 == 7) & (j == nblk - 1)
        fix_at = _SEQ_BLK - 8
    else:
        fixmask = (iota8 == 0) & (j == 0)
        fix_at = _HALO
    nbedge = -b_edge
    zrow = jnp.zeros((1, d), dtype=h.dtype)

    wc1 = wc1_ref[...]
    wc2 = wc2_ref[...]

    m = h.shape[0]
    step = ((m // _NCHUNK) // 8) * 8
    bounds = [0] + [step * i for i in range(1, _NCHUNK)] + [m]
    hs = [h[a:b] for a, b in zip(bounds[:-1], bounds[1:])]
    nc = _NCHUNK

    def _update(hc, pp, q):
        upd = jnp.tanh(q[:, :d] + b2_msg)
        gate = jax.nn.sigmoid(pp[:, d:] + (q[:, d:] + b2_gate))
        return hc + gate * upd

    for _ in range(_TREE_ITERS):
        ps = [jnp.dot(hc.astype(jnp.bfloat16), wc1,
                      preferred_element_type=jnp.float32) for hc in hs]
        pms = [pp[:, :d] for pp in ps]
        aggs = []
        if reverse:
            for c in range(nc):
                nxt = zrow if c == nc - 1 else pms[c + 1][:1]
                if c == nc - 1:
                    off = fix_at - bounds[c]
                    sl = jnp.where(fixmask, nbedge, pms[c][off + 1:off + 9])
                    aggs.append(jnp.concatenate(
                        [pms[c][1:off + 1], sl, pms[c][off + 9:], nxt],
                        axis=0))
                else:
                    aggs.append(jnp.concatenate([pms[c][1:], nxt], axis=0))
        else:
            for c in range(nc):
                prv = zrow if c == 0 else pms[c - 1][-1:]
                if c == 0:
                    sl = jnp.where(fixmask, nbedge,
                                   pms[0][fix_at - 1:fix_at + 7])
                    aggs.append(jnp.concatenate(
                        [prv, pms[0][:fix_at - 1], sl, pms[0][fix_at + 7:-1]],
                        axis=0))
                else:
                    aggs.append(jnp.concatenate([prv, pms[c][:-1]], axis=0))
        qs = []
        new_hs = []
        for c in range(nc):
            qs.append(jnp.dot(aggs[c].astype(jnp.bfloat16), wc2,
                              preferred_element_type=jnp.float32))
            if c > 0:
                new_hs.append(_update(hs[c - 1], ps[c - 1], qs[c - 1]))
        new_hs.append(_update(hs[nc - 1], ps[nc - 1], qs[nc - 1]))
        hs = new_hs

    h = jnp.concatenate(hs, axis=0)
    if reverse:
        out_ref[0] = x_blk + h[:_SEQ_BLK]
    else:
        out_ref[0] = x_blk + h[_HALO:]


def _phase(x_in, p, reverse):
    b, s, d = x_in.shape
    wc1f = jnp.concatenate([p["W_edge"], p["W_gate"][:d]], axis=1)
    wc2f = jnp.concatenate([p["W_msg"], p["W_gate"][d:]], axis=1)
    wc1 = wc1f.astype(jnp.bfloat16)
    wc2 = wc2f.astype(jnp.bfloat16)
    bvec = p["b_edge"] @ wc2f
    zero = jnp.zeros_like(p["b_edge"])
    bias = jnp.stack([p["b_edge"], p["b_msg"] + bvec[:d],
                      p["b_gate"] + bvec[d:],
                      p["ln_g"], p["ln_b"], zero, zero, zero])

    nblk = s // _SEQ_BLK
    hb = _SEQ_BLK // _HALO
    last_halo_blk = s // _HALO - 1

    if reverse:
        def halo_map(bi, ji):
            return (bi, jnp.minimum((ji + 1) * hb, last_halo_blk), 0)
    else:
        def halo_map(bi, ji):
            return (bi, jnp.maximum(ji * hb - 1, 0), 0)

    return pl.pallas_call(
        functools.partial(_phase_kernel, reverse=reverse, nblk=nblk),
        grid=(b, nblk),
        in_specs=[
            pl.BlockSpec((1, _SEQ_BLK, d), lambda bi, ji: (bi, ji, 0)),
            pl.BlockSpec((1, _HALO, d), halo_map),
            pl.BlockSpec((d, 2 * d), lambda bi, ji: (0, 0)),
            pl.BlockSpec((d, 2 * d), lambda bi, ji: (0, 0)),
            pl.BlockSpec((8, d), lambda bi, ji: (0, 0)),
        ],
        out_specs=pl.BlockSpec((1, _SEQ_BLK, d), lambda bi, ji: (bi, ji, 0)),
        out_shape=jax.ShapeDtypeStruct((b, s, d), x_in.dtype),
    )(x_in, x_in, wc1, wc2, bias)


def kernel(x, params):
    h = _phase(x, params["enc"], reverse=False)
    h = _phase(h, params["dec"], reverse=True)
    return h
